# Initial kernel scaffold; baseline (speedup 1.0000x reference)
import jax
import jax.numpy as jnp
from jax.experimental import pallas as pl


def kernel(x, edge_index0, edge_weight0, edge_index1, edge_weight1, W1, b1, W2, b2):
    raise NotImplementedError("write your pallas kernel here")


# speedup vs baseline: 4.8422x; 4.8422x over previous
import functools

import jax
import jax.numpy as jnp
from jax import lax
from jax.experimental import pallas as pl
from jax.experimental.pallas import tpu as pltpu
from jax.experimental.pallas import tpu_sc as plsc

N = 10000
NP = 10240
E = 320000
NC = 2
NS = 16
NW = NC * NS
CHUNK = 128
CPW = 80
EPW = E // NW
RPS = NP // NS
FIN = 128
HID = 64

_MESH = plsc.VectorSubcoreMesh(core_axis_name="c", subcore_axis_name="s")

_SC_PARAMS = pltpu.CompilerParams(
    needs_layout_passes=False,
    use_tc_tiling_on_sc=False,
)


def _splat(v):
    return jnp.full((16,), v, jnp.int32)


@functools.partial(
    pl.kernel,
    out_type=jax.ShapeDtypeStruct((NC, 4, NP), jnp.float32),
    mesh=_MESH,
    compiler_params=_SC_PARAMS,
    scratch_types=[
        pltpu.VMEM((CPW, CHUNK), jnp.int32),
        pltpu.VMEM((CHUNK,), jnp.float32),
        pltpu.VMEM((RPS,), jnp.float32),
        pltpu.VMEM_SHARED((NP,), jnp.float32),
        pltpu.VMEM_SHARED((NP,), jnp.float32),
        pltpu.VMEM_SHARED((NP,), jnp.float32),
        pltpu.VMEM_SHARED((NP,), jnp.float32),
        pltpu.SemaphoreType.DMA,
    ],
)
def _deg_kernel(idx_hbm, out_hbm, idx_v, ones_v, zrow_v, d0, d1, d2, d3, sem):
    c = lax.axis_index("c")
    s = lax.axis_index("s")
    w = c * NS + s
    tables = (d0, d1, d2, d3)

    @pl.loop(0, CHUNK, step=16)
    def _(i):
        ones_v[pl.ds(i, 16)] = jnp.ones((16,), jnp.float32)

    @pl.loop(0, RPS, step=16)
    def _(i):
        zrow_v[pl.ds(i, 16)] = jnp.zeros((16,), jnp.float32)

    row0 = s * RPS
    for t in range(4):
        pltpu.sync_copy(zrow_v, tables[t].at[pl.ds(row0, RPS)])
    plsc.subcore_barrier()

    for t in range(4):
        pltpu.sync_copy(idx_hbm.at[t, w], idx_v)
        for j0 in range(0, CPW, 16):
            descs = [
                pltpu.async_copy(ones_v, tables[t].at[idx_v.at[j]], sem, add=True)
                for j in range(j0, j0 + 16)
            ]
            for d in descs:
                d.wait()

    plsc.subcore_barrier()
    for t in range(4):
        pltpu.sync_copy(tables[t].at[pl.ds(row0, RPS)],
                        out_hbm.at[c, t, pl.ds(row0, RPS)])


@functools.partial(
    pl.kernel,
    out_type=jax.ShapeDtypeStruct((NC, NP, HID), jnp.float32),
    mesh=_MESH,
    compiler_params=_SC_PARAMS,
    scratch_types=[
        pltpu.VMEM((CPW, CHUNK), jnp.int32),
        pltpu.VMEM((CPW, CHUNK), jnp.int32),
        pltpu.VMEM((CPW, CHUNK), jnp.float32),
        pltpu.VMEM((CPW, CHUNK), jnp.float32),
        pltpu.VMEM((NP,), jnp.float32),
        pltpu.VMEM((NP,), jnp.float32),
        pltpu.VMEM((CHUNK, HID), jnp.float32),
        pltpu.VMEM((64, HID), jnp.float32),
        pltpu.VMEM_SHARED((NP, HID), jnp.float32),
        pltpu.SemaphoreType.DMA,
    ],
)
def _layer_kernel(t_hbm, src_hbm, dst_hbm, ew_hbm, ns_hbm, nd_hbm, out_hbm,
                  src_v, dst_v, ew_v, c_v, ns_v, nd_v, rows_v, zb_v, agg_sh,
                  sem):
    c = lax.axis_index("c")
    s = lax.axis_index("s")
    w = c * NS + s

    pltpu.sync_copy(src_hbm.at[w], src_v)
    pltpu.sync_copy(dst_hbm.at[w], dst_v)
    pltpu.sync_copy(ew_hbm.at[w], ew_v)
    pltpu.sync_copy(ns_hbm, ns_v)
    pltpu.sync_copy(nd_hbm, nd_v)

    @pl.loop(0, 64)
    def _(r):
        for q in range(HID // 16):
            zb_v[r, pl.ds(q * 16, 16)] = jnp.zeros((16,), jnp.float32)

    for r in range(RPS // 64):
        pltpu.sync_copy(zb_v, agg_sh.at[pl.ds(s * RPS + r * 64, 64)])

    @pl.loop(0, CPW)
    def _(j):
        @pl.loop(0, CHUNK, step=16)
        def _(e):
            sl = pl.ds(e, 16)
            si = src_v[j, sl]
            di = dst_v[j, sl]
            c_v[j, sl] = (ew_v[j, sl]
                          * plsc.load_gather(ns_v, [si])
                          * plsc.load_gather(nd_v, [di]))

    plsc.subcore_barrier()

    @pl.loop(0, CPW)
    def _(j):
        pltpu.async_copy(t_hbm.at[src_v.at[j]], rows_v, sem).wait()

        @pl.loop(0, CHUNK, unroll=8)
        def _(e):
            w16 = plsc.load_gather(c_v, [_splat(j), _splat(e)])
            for q in range(HID // 16):
                sl = pl.ds(q * 16, 16)
                rows_v[e, sl] = rows_v[e, sl] * w16

        pltpu.sync_copy(rows_v, agg_sh.at[dst_v.at[j]], add=True)

    plsc.subcore_barrier()
    pltpu.sync_copy(agg_sh.at[pl.ds(s * RPS, RPS)],
                    out_hbm.at[c].at[pl.ds(s * RPS, RPS)])


def _tc_norms_mm(degp, xp, w1):
    def body(degp_ref, x_ref, w_ref, norms_ref, t1_ref):
        deg = degp_ref[0] + degp_ref[1]
        norms_ref[...] = lax.rsqrt(jnp.where(deg > 0.0, deg, 1.0))
        t1_ref[...] = jnp.dot(x_ref[...], w_ref[...],
                              preferred_element_type=jnp.float32,
                              precision=lax.Precision.HIGHEST)

    return pl.pallas_call(
        body,
        out_shape=(jax.ShapeDtypeStruct((4, NP), jnp.float32),
                   jax.ShapeDtypeStruct((NP, HID), jnp.float32)),
    )(degp, xp, w1)


def _tc_relu_mm(aggp, b, w2):
    def body(aggp_ref, b_ref, w_ref, t2_ref):
        h = jnp.maximum(aggp_ref[0] + aggp_ref[1] + b_ref[...][None, :], 0.0)
        t2_ref[...] = jnp.dot(h, w_ref[...],
                              preferred_element_type=jnp.float32,
                              precision=lax.Precision.HIGHEST)

    return pl.pallas_call(
        body,
        out_shape=jax.ShapeDtypeStruct((NP, HID), jnp.float32),
    )(aggp, b, w2)


def _tc_relu_out(aggp, b):
    def body(aggp_ref, b_ref, o_ref):
        h = aggp_ref[0] + aggp_ref[1] + b_ref[...][None, :]
        o_ref[...] = jnp.maximum(h[:N], 0.0)

    return pl.pallas_call(
        body,
        out_shape=jax.ShapeDtypeStruct((N, HID), jnp.float32),
    )(aggp, b)


def _pad_idx(a):
    a = a.reshape(NW, EPW)
    a = jnp.pad(a, ((0, 0), (0, CPW * CHUNK - EPW)), constant_values=N)
    return a.reshape(NW, CPW, CHUNK)


def _pad_ew(a):
    a = a.reshape(NW, EPW)
    a = jnp.pad(a, ((0, 0), (0, CPW * CHUNK - EPW)))
    return a.reshape(NW, CPW, CHUNK)


def kernel(x, edge_index0, edge_weight0, edge_index1, edge_weight1,
           W1, b1, W2, b2):
    src0 = _pad_idx(edge_index0[0])
    dst0 = _pad_idx(edge_index0[1])
    src1 = _pad_idx(edge_index1[0])
    dst1 = _pad_idx(edge_index1[1])
    ew0 = _pad_ew(edge_weight0)
    ew1 = _pad_ew(edge_weight1)
    xp = jnp.pad(x, ((0, NP - N), (0, 0)))

    idxs = jnp.stack([src0, dst0, src1, dst1])
    degp = _deg_kernel(idxs)

    norms, t1 = _tc_norms_mm(degp, xp, W1)
    aggp0 = _layer_kernel(t1, src0, dst0, ew0, norms[0], norms[1])
    t2 = _tc_relu_mm(aggp0, b1, W2)
    aggp1 = _layer_kernel(t2, src1, dst1, ew1, norms[2], norms[3])
    return _tc_relu_out(aggp1, b2)


# speedup vs baseline: 6.2899x; 1.2990x over previous
import functools

import jax
import jax.numpy as jnp
from jax import lax
from jax.experimental import pallas as pl
from jax.experimental.pallas import tpu as pltpu
from jax.experimental.pallas import tpu_sc as plsc

N = 10000
NP = 10240
E = 320000
NC = 2
NS = 16
NW = NC * NS
CHUNK = 128
CPW = 80
EPW = E // NW
RPS = NP // NS
FIN = 128
HID = 64

_MESH = plsc.VectorSubcoreMesh(core_axis_name="c", subcore_axis_name="s")

_SC_PARAMS = pltpu.CompilerParams(
    needs_layout_passes=False,
    use_tc_tiling_on_sc=False,
)


def _splat(v):
    return jnp.full((16,), v, jnp.int32)


@functools.partial(
    pl.kernel,
    out_type=jax.ShapeDtypeStruct((NC, 4, NP), jnp.float32),
    mesh=_MESH,
    compiler_params=_SC_PARAMS,
    scratch_types=[
        pltpu.VMEM((CPW, CHUNK), jnp.int32),
        pltpu.VMEM((CHUNK,), jnp.float32),
        pltpu.VMEM((RPS,), jnp.float32),
        pltpu.VMEM_SHARED((NP,), jnp.float32),
        pltpu.VMEM_SHARED((NP,), jnp.float32),
        pltpu.VMEM_SHARED((NP,), jnp.float32),
        pltpu.VMEM_SHARED((NP,), jnp.float32),
        pltpu.SemaphoreType.DMA,
    ],
)
def _deg_kernel(idx_hbm, out_hbm, idx_v, ones_v, zrow_v, d0, d1, d2, d3, sem):
    c = lax.axis_index("c")
    s = lax.axis_index("s")
    w = c * NS + s
    tables = (d0, d1, d2, d3)

    @pl.loop(0, CHUNK, step=16)
    def _(i):
        ones_v[pl.ds(i, 16)] = jnp.ones((16,), jnp.float32)

    @pl.loop(0, RPS, step=16)
    def _(i):
        zrow_v[pl.ds(i, 16)] = jnp.zeros((16,), jnp.float32)

    row0 = s * RPS
    for t in range(4):
        pltpu.sync_copy(zrow_v, tables[t].at[pl.ds(row0, RPS)])
    plsc.subcore_barrier()

    for t in range(4):
        pltpu.sync_copy(idx_hbm.at[t, w], idx_v)
        for j0 in range(0, CPW, 16):
            descs = [
                pltpu.async_copy(ones_v, tables[t].at[idx_v.at[j]], sem, add=True)
                for j in range(j0, j0 + 16)
            ]
            for d in descs:
                d.wait()

    plsc.subcore_barrier()
    for t in range(4):
        pltpu.sync_copy(tables[t].at[pl.ds(row0, RPS)],
                        out_hbm.at[c, t, pl.ds(row0, RPS)])


@functools.partial(
    pl.kernel,
    out_type=jax.ShapeDtypeStruct((NC, NP, HID), jnp.float32),
    mesh=_MESH,
    compiler_params=_SC_PARAMS,
    scratch_types=[
        pltpu.VMEM((CPW, CHUNK), jnp.int32),
        pltpu.VMEM((CPW, CHUNK), jnp.int32),
        pltpu.VMEM((CPW, CHUNK), jnp.float32),
        pltpu.VMEM((NP,), jnp.float32),
        pltpu.VMEM((NP,), jnp.float32),
        pltpu.VMEM((4, CHUNK, HID), jnp.float32),
        pltpu.VMEM((64, HID), jnp.float32),
        pltpu.VMEM_SHARED((NP, HID), jnp.float32),
        pltpu.SemaphoreType.DMA,
        pltpu.SemaphoreType.DMA,
        pltpu.SemaphoreType.DMA,
        pltpu.SemaphoreType.DMA,
        pltpu.SemaphoreType.DMA,
        pltpu.SemaphoreType.DMA,
        pltpu.SemaphoreType.DMA,
        pltpu.SemaphoreType.DMA,
    ],
)
def _layer_kernel(t_hbm, src_hbm, dst_hbm, ew_hbm, ns_hbm, nd_hbm, out_hbm,
                  src_v, dst_v, c_v, ns_v, nd_v, rows_v, zb_v, agg_sh,
                  gs0, gs1, gs2, gs3, ss0, ss1, ss2, ss3):
    c = lax.axis_index("c")
    s = lax.axis_index("s")
    w = c * NS + s

    pltpu.sync_copy(src_hbm.at[w], src_v)
    pltpu.sync_copy(dst_hbm.at[w], dst_v)
    pltpu.sync_copy(ew_hbm.at[w], c_v)
    pltpu.sync_copy(ns_hbm, ns_v)
    pltpu.sync_copy(nd_hbm, nd_v)

    @pl.loop(0, 64)
    def _(r):
        for q in range(HID // 16):
            zb_v[r, pl.ds(q * 16, 16)] = jnp.zeros((16,), jnp.float32)

    for r in range(RPS // 64):
        pltpu.sync_copy(zb_v, agg_sh.at[pl.ds(s * RPS + r * 64, 64)])

    @pl.loop(0, CPW)
    def _(j):
        @pl.loop(0, CHUNK, step=16)
        def _(e):
            sl = pl.ds(e, 16)
            si = src_v[j, sl]
            di = dst_v[j, sl]
            c_v[j, sl] = (c_v[j, sl]
                          * plsc.load_gather(ns_v, [si])
                          * plsc.load_gather(nd_v, [di]))

    plsc.subcore_barrier()

    bufs = (rows_v.at[0], rows_v.at[1], rows_v.at[2], rows_v.at[3])
    gsems = (gs0, gs1, gs2, gs3)
    ssems = (ss0, ss1, ss2, ss3)

    pltpu.async_copy(t_hbm.at[src_v.at[0]], bufs[0], gsems[0])
    pltpu.async_copy(t_hbm.at[src_v.at[1]], bufs[1], gsems[1])

    @pl.loop(0, CPW, step=4)
    def _(j):
        for b in range(4):
            jj = j + b
            bp = (b + 2) % 4
            jp = jj + 2

            @pl.when(jnp.logical_and(jp < CPW, jp >= 4))
            def _():
                pltpu.make_async_copy(
                    bufs[bp], agg_sh.at[dst_v.at[jp - 4]], ssems[bp]).wait()

            @pl.when(jp < CPW)
            def _():
                pltpu.async_copy(t_hbm.at[src_v.at[jp]], bufs[bp], gsems[bp])

            pltpu.make_async_copy(
                t_hbm.at[src_v.at[jj]], bufs[b], gsems[b]).wait()

            @pl.loop(0, CHUNK, unroll=8)
            def _(e):
                w16 = plsc.load_gather(c_v, [_splat(jj), _splat(e)])
                for q in range(HID // 16):
                    sl = pl.ds(q * 16, 16)
                    bufs[b][e, sl] = bufs[b][e, sl] * w16

            pltpu.async_copy(bufs[b], agg_sh.at[dst_v.at[jj]], ssems[b],
                             add=True)

    for b in range(4):
        pltpu.make_async_copy(
            bufs[b], agg_sh.at[dst_v.at[CPW - 4 + b]], ssems[b]).wait()

    plsc.subcore_barrier()
    pltpu.sync_copy(agg_sh.at[pl.ds(s * RPS, RPS)],
                    out_hbm.at[c].at[pl.ds(s * RPS, RPS)])


def _tc_norms_mm(degp, xp, w1):
    def body(degp_ref, x_ref, w_ref, norms_ref, t1_ref):
        deg = degp_ref[0] + degp_ref[1]
        norms_ref[...] = lax.rsqrt(jnp.where(deg > 0.0, deg, 1.0))
        t1_ref[...] = jnp.dot(x_ref[...], w_ref[...],
                              preferred_element_type=jnp.float32,
                              precision=lax.Precision.HIGHEST)

    return pl.pallas_call(
        body,
        out_shape=(jax.ShapeDtypeStruct((4, NP), jnp.float32),
                   jax.ShapeDtypeStruct((NP, HID), jnp.float32)),
    )(degp, xp, w1)


def _tc_relu_mm(aggp, b, w2):
    def body(aggp_ref, b_ref, w_ref, t2_ref):
        h = jnp.maximum(aggp_ref[0] + aggp_ref[1] + b_ref[...][None, :], 0.0)
        t2_ref[...] = jnp.dot(h, w_ref[...],
                              preferred_element_type=jnp.float32,
                              precision=lax.Precision.HIGHEST)

    return pl.pallas_call(
        body,
        out_shape=jax.ShapeDtypeStruct((NP, HID), jnp.float32),
    )(aggp, b, w2)


def _tc_relu_out(aggp, b):
    def body(aggp_ref, b_ref, o_ref):
        h = aggp_ref[0] + aggp_ref[1] + b_ref[...][None, :]
        o_ref[...] = jnp.maximum(h[:N], 0.0)

    return pl.pallas_call(
        body,
        out_shape=jax.ShapeDtypeStruct((N, HID), jnp.float32),
    )(aggp, b)


def _pad_idx(a):
    a = a.reshape(NW, EPW)
    a = jnp.pad(a, ((0, 0), (0, CPW * CHUNK - EPW)), constant_values=N)
    return a.reshape(NW, CPW, CHUNK)


def _pad_ew(a):
    a = a.reshape(NW, EPW)
    a = jnp.pad(a, ((0, 0), (0, CPW * CHUNK - EPW)))
    return a.reshape(NW, CPW, CHUNK)


def kernel(x, edge_index0, edge_weight0, edge_index1, edge_weight1,
           W1, b1, W2, b2):
    src0 = _pad_idx(edge_index0[0])
    dst0 = _pad_idx(edge_index0[1])
    src1 = _pad_idx(edge_index1[0])
    dst1 = _pad_idx(edge_index1[1])
    ew0 = _pad_ew(edge_weight0)
    ew1 = _pad_ew(edge_weight1)
    xp = jnp.pad(x, ((0, NP - N), (0, 0)))

    idxs = jnp.stack([src0, dst0, src1, dst1])
    degp = _deg_kernel(idxs)

    norms, t1 = _tc_norms_mm(degp, xp, W1)
    aggp0 = _layer_kernel(t1, src0, dst0, ew0, norms[0], norms[1])
    t2 = _tc_relu_mm(aggp0, b1, W2)
    aggp1 = _layer_kernel(t2, src1, dst1, ew1, norms[2], norms[3])
    return _tc_relu_out(aggp1, b2)


# speedup vs baseline: 6.8021x; 1.0814x over previous
import functools

import jax
import jax.numpy as jnp
from jax import lax
from jax.experimental import pallas as pl
from jax.experimental.pallas import tpu as pltpu
from jax.experimental.pallas import tpu_sc as plsc

N = 10000
NP = 10240
E = 320000
NC = 2
NS = 16
NW = NC * NS
CHUNK = 128
CPW = 80
EPW = E // NW
RPS = NP // NS
FIN = 128
HID = 64

_MESH = plsc.VectorSubcoreMesh(core_axis_name="c", subcore_axis_name="s")

_SC_PARAMS = pltpu.CompilerParams(
    needs_layout_passes=False,
    use_tc_tiling_on_sc=False,
)


def _splat(v):
    return jnp.full((16,), v, jnp.int32)


@functools.partial(
    pl.kernel,
    out_type=jax.ShapeDtypeStruct((NC, 4, NP), jnp.float32),
    mesh=_MESH,
    compiler_params=_SC_PARAMS,
    scratch_types=[
        pltpu.VMEM((CPW, CHUNK), jnp.int32),
        pltpu.VMEM((CHUNK,), jnp.float32),
        pltpu.VMEM((RPS,), jnp.float32),
        pltpu.VMEM_SHARED((NP,), jnp.float32),
        pltpu.VMEM_SHARED((NP,), jnp.float32),
        pltpu.VMEM_SHARED((NP,), jnp.float32),
        pltpu.VMEM_SHARED((NP,), jnp.float32),
        pltpu.SemaphoreType.DMA,
    ],
)
def _deg_kernel(idx_hbm, out_hbm, idx_v, ones_v, zrow_v, d0, d1, d2, d3, sem):
    c = lax.axis_index("c")
    s = lax.axis_index("s")
    w = c * NS + s
    tables = (d0, d1, d2, d3)

    @pl.loop(0, CHUNK, step=16)
    def _(i):
        ones_v[pl.ds(i, 16)] = jnp.ones((16,), jnp.float32)

    @pl.loop(0, RPS, step=16)
    def _(i):
        zrow_v[pl.ds(i, 16)] = jnp.zeros((16,), jnp.float32)

    row0 = s * RPS
    for t in range(4):
        pltpu.sync_copy(zrow_v, tables[t].at[pl.ds(row0, RPS)])
    plsc.subcore_barrier()

    for t in range(4):
        pltpu.sync_copy(idx_hbm.at[t, w], idx_v)
        for j0 in range(0, CPW, 16):
            descs = [
                pltpu.async_copy(ones_v, tables[t].at[idx_v.at[j]], sem, add=True)
                for j in range(j0, j0 + 16)
            ]
            for d in descs:
                d.wait()

    plsc.subcore_barrier()
    for t in range(4):
        pltpu.sync_copy(tables[t].at[pl.ds(row0, RPS)],
                        out_hbm.at[c, t, pl.ds(row0, RPS)])


_EVEN = tuple(range(0, 32, 2))
_ODD = tuple(range(1, 32, 2))


@functools.partial(
    pl.kernel,
    out_type=jax.ShapeDtypeStruct((NC, NP, HID), jnp.float32),
    mesh=_MESH,
    compiler_params=_SC_PARAMS,
    scratch_types=[
        pltpu.VMEM((CPW, CHUNK), jnp.int32),
        pltpu.VMEM((CPW, CHUNK), jnp.int32),
        pltpu.VMEM((CPW, CHUNK), jnp.float32),
        pltpu.VMEM((NP,), jnp.float32),
        pltpu.VMEM((NP,), jnp.float32),
        pltpu.VMEM((2, CHUNK, HID), jnp.bfloat16),
        pltpu.VMEM((2, CHUNK, HID), jnp.float32),
        pltpu.VMEM_SHARED((NP, HID), jnp.float32),
        pltpu.SemaphoreType.DMA,
        pltpu.SemaphoreType.DMA,
        pltpu.SemaphoreType.DMA,
        pltpu.SemaphoreType.DMA,
    ],
)
def _layer_kernel(t_hbm, src_hbm, dst_hbm, ew_hbm, ns_hbm, nd_hbm, out_hbm,
                  src_v, dst_v, c_v, ns_v, nd_v, grow_v, srow_v, agg_sh,
                  gs0, gs1, ss0, ss1):
    c = lax.axis_index("c")
    s = lax.axis_index("s")
    w = c * NS + s

    pltpu.sync_copy(src_hbm.at[w], src_v)
    pltpu.sync_copy(dst_hbm.at[w], dst_v)
    pltpu.sync_copy(ew_hbm.at[w], c_v)
    pltpu.sync_copy(ns_hbm, ns_v)
    pltpu.sync_copy(nd_hbm, nd_v)

    @pl.loop(0, CHUNK)
    def _(r):
        for q in range(HID // 16):
            srow_v[0, r, pl.ds(q * 16, 16)] = jnp.zeros((16,), jnp.float32)

    for r in range(RPS // CHUNK):
        pltpu.sync_copy(srow_v.at[0],
                        agg_sh.at[pl.ds(s * RPS + r * CHUNK, CHUNK)])

    @pl.loop(0, CPW)
    def _(j):
        @pl.loop(0, CHUNK, step=16)
        def _(e):
            sl = pl.ds(e, 16)
            si = src_v[j, sl]
            di = dst_v[j, sl]
            c_v[j, sl] = (c_v[j, sl]
                          * plsc.load_gather(ns_v, [si])
                          * plsc.load_gather(nd_v, [di]))

    plsc.subcore_barrier()

    gbufs = (grow_v.at[0], grow_v.at[1])
    sbufs = (srow_v.at[0], srow_v.at[1])
    gsems = (gs0, gs1)
    ssems = (ss0, ss1)

    iota16 = lax.iota(jnp.int32, 16)
    idx_even = iota16 * 2
    idx_odd = iota16 * 2 + 1

    pltpu.async_copy(t_hbm.at[src_v.at[0]], gbufs[0], gsems[0])
    pltpu.async_copy(t_hbm.at[src_v.at[1]], gbufs[1], gsems[1])

    @pl.loop(0, CPW, step=2)
    def _(j):
        for b in range(2):
            jj = j + b

            @pl.when(jj >= 2)
            def _():
                pltpu.make_async_copy(
                    sbufs[b], agg_sh.at[dst_v.at[jj - 2]], ssems[b]).wait()

            pltpu.make_async_copy(
                t_hbm.at[src_v.at[jj]], gbufs[b], gsems[b]).wait()

            @pl.loop(0, CHUNK, unroll=8)
            def _(e):
                w16 = plsc.load_gather(c_v, [_splat(jj), _splat(e)])
                for h in range(2):
                    v32 = gbufs[b][e, pl.ds(h * 32, 32)]
                    wi = plsc.bitcast(v32, jnp.int32)
                    lo = plsc.bitcast(wi << 16, jnp.float32) * w16
                    hi = plsc.bitcast(wi & jnp.int32(-65536), jnp.float32) * w16
                    plsc.store_scatter(srow_v,
                                       [_splat(b), _splat(e), idx_even + h * 32],
                                       lo)
                    plsc.store_scatter(srow_v,
                                       [_splat(b), _splat(e), idx_odd + h * 32],
                                       hi)

            pltpu.async_copy(sbufs[b], agg_sh.at[dst_v.at[jj]], ssems[b],
                             add=True)

            @pl.when(jj + 2 < CPW)
            def _():
                pltpu.async_copy(t_hbm.at[src_v.at[jj + 2]], gbufs[b],
                                 gsems[b])

    for b in range(2):
        pltpu.make_async_copy(
            sbufs[b], agg_sh.at[dst_v.at[CPW - 2 + b]], ssems[b]).wait()

    plsc.subcore_barrier()
    pltpu.sync_copy(agg_sh.at[pl.ds(s * RPS, RPS)],
                    out_hbm.at[c].at[pl.ds(s * RPS, RPS)])


def _tc_norms_mm(degp, xp, w1):
    def body(degp_ref, x_ref, w_ref, norms_ref, t1_ref):
        deg = degp_ref[0] + degp_ref[1]
        norms_ref[...] = lax.rsqrt(jnp.where(deg > 0.0, deg, 1.0))
        t1_ref[...] = jnp.dot(x_ref[...], w_ref[...],
                              preferred_element_type=jnp.float32,
                              precision=lax.Precision.HIGHEST
                              ).astype(jnp.bfloat16)

    return pl.pallas_call(
        body,
        out_shape=(jax.ShapeDtypeStruct((4, NP), jnp.float32),
                   jax.ShapeDtypeStruct((NP, HID), jnp.bfloat16)),
    )(degp, xp, w1)


def _tc_relu_mm(aggp, b, w2):
    def body(aggp_ref, b_ref, w_ref, t2_ref):
        h = jnp.maximum(aggp_ref[0] + aggp_ref[1] + b_ref[...][None, :], 0.0)
        t2_ref[...] = jnp.dot(h, w_ref[...],
                              preferred_element_type=jnp.float32,
                              precision=lax.Precision.HIGHEST
                              ).astype(jnp.bfloat16)

    return pl.pallas_call(
        body,
        out_shape=jax.ShapeDtypeStruct((NP, HID), jnp.bfloat16),
    )(aggp, b, w2)


def _tc_relu_out(aggp, b):
    def body(aggp_ref, b_ref, o_ref):
        h = aggp_ref[0] + aggp_ref[1] + b_ref[...][None, :]
        o_ref[...] = jnp.maximum(h[:N], 0.0)

    return pl.pallas_call(
        body,
        out_shape=jax.ShapeDtypeStruct((N, HID), jnp.float32),
    )(aggp, b)


def _pad_idx(a):
    a = a.reshape(NW, EPW)
    a = jnp.pad(a, ((0, 0), (0, CPW * CHUNK - EPW)), constant_values=N)
    return a.reshape(NW, CPW, CHUNK)


def _pad_ew(a):
    a = a.reshape(NW, EPW)
    a = jnp.pad(a, ((0, 0), (0, CPW * CHUNK - EPW)))
    return a.reshape(NW, CPW, CHUNK)


def kernel(x, edge_index0, edge_weight0, edge_index1, edge_weight1,
           W1, b1, W2, b2):
    src0 = _pad_idx(edge_index0[0])
    dst0 = _pad_idx(edge_index0[1])
    src1 = _pad_idx(edge_index1[0])
    dst1 = _pad_idx(edge_index1[1])
    ew0 = _pad_ew(edge_weight0)
    ew1 = _pad_ew(edge_weight1)
    xp = jnp.pad(x, ((0, NP - N), (0, 0)))

    idxs = jnp.stack([src0, dst0, src1, dst1])
    degp = _deg_kernel(idxs)

    norms, t1 = _tc_norms_mm(degp, xp, W1)
    aggp0 = _layer_kernel(t1, src0, dst0, ew0, norms[0], norms[1])
    t2 = _tc_relu_mm(aggp0, b1, W2)
    aggp1 = _layer_kernel(t2, src1, dst1, ew1, norms[2], norms[3])
    return _tc_relu_out(aggp1, b2)


# speedup vs baseline: 7.0330x; 1.0339x over previous
import functools

import jax
import jax.numpy as jnp
from jax import lax
from jax.experimental import pallas as pl
from jax.experimental.pallas import tpu as pltpu
from jax.experimental.pallas import tpu_sc as plsc

N = 10000
NP = 10240
E = 320000
NC = 2
NS = 16
NW = NC * NS
CHUNK = 128
CPW = 80
EPW = E // NW
RPS = NP // NS
FIN = 128
HID = 64

_MESH = plsc.VectorSubcoreMesh(core_axis_name="c", subcore_axis_name="s")

_SC_PARAMS = pltpu.CompilerParams(
    needs_layout_passes=False,
    use_tc_tiling_on_sc=False,
)


def _splat(v):
    return jnp.full((16,), v, jnp.int32)


@functools.partial(
    pl.kernel,
    out_type=jax.ShapeDtypeStruct((4, NP), jnp.float32),
    mesh=_MESH,
    compiler_params=_SC_PARAMS,
    scratch_types=[
        pltpu.VMEM((CPW, CHUNK), jnp.int32),
        pltpu.VMEM((CHUNK,), jnp.float32),
        pltpu.VMEM((RPS,), jnp.float32),
        pltpu.VMEM_SHARED((NP,), jnp.float32),
        pltpu.VMEM_SHARED((NP,), jnp.float32),
        pltpu.SemaphoreType.DMA,
    ],
)
def _deg_kernel(idx_hbm, out_hbm, idx_v, ones_v, zrow_v, d0, d1, sem):
    c = lax.axis_index("c")
    s = lax.axis_index("s")
    tables = (d0, d1)

    @pl.loop(0, CHUNK, step=16)
    def _(i):
        ones_v[pl.ds(i, 16)] = jnp.ones((16,), jnp.float32)

    @pl.loop(0, RPS, step=16)
    def _(i):
        zrow_v[pl.ds(i, 16)] = jnp.zeros((16,), jnp.float32)

    row0 = s * RPS
    for k in range(2):
        pltpu.sync_copy(zrow_v, tables[k].at[pl.ds(row0, RPS)])
    plsc.subcore_barrier()

    for k in range(2):
        for half in range(2):
            pltpu.sync_copy(idx_hbm.at[2 * c + k, s + half * NS], idx_v)
            for j0 in range(0, CPW, 16):
                descs = [
                    pltpu.async_copy(ones_v, tables[k].at[idx_v.at[j]], sem,
                                     add=True)
                    for j in range(j0, j0 + 16)
                ]
                for d in descs:
                    d.wait()

    plsc.subcore_barrier()
    for k in range(2):
        pltpu.sync_copy(tables[k].at[pl.ds(row0, RPS)], zrow_v)

        @pl.loop(0, RPS, step=16)
        def _(i):
            sl = pl.ds(i, 16)
            x = jnp.maximum(zrow_v[sl], 1.0)
            xi = plsc.bitcast(x, jnp.int32)
            y = plsc.bitcast(jnp.int32(0x5F3759DF) - (xi >> 1), jnp.float32)
            for _ in range(3):
                y = y * (1.5 - 0.5 * x * y * y)
            zrow_v[sl] = y

        pltpu.sync_copy(zrow_v, out_hbm.at[2 * c + k, pl.ds(row0, RPS)])


_EVEN = tuple(range(0, 32, 2))
_ODD = tuple(range(1, 32, 2))


@functools.partial(
    pl.kernel,
    out_type=jax.ShapeDtypeStruct((NC, NP, HID), jnp.float32),
    mesh=_MESH,
    compiler_params=_SC_PARAMS,
    scratch_types=[
        pltpu.VMEM((CPW, CHUNK), jnp.int32),
        pltpu.VMEM((CPW, CHUNK), jnp.int32),
        pltpu.VMEM((CPW, CHUNK), jnp.float32),
        pltpu.VMEM((NP,), jnp.float32),
        pltpu.VMEM((NP,), jnp.float32),
        pltpu.VMEM((2, CHUNK, HID), jnp.bfloat16),
        pltpu.VMEM((2, CHUNK, HID), jnp.float32),
        pltpu.VMEM_SHARED((NP, HID), jnp.float32),
        pltpu.SemaphoreType.DMA,
        pltpu.SemaphoreType.DMA,
        pltpu.SemaphoreType.DMA,
        pltpu.SemaphoreType.DMA,
    ],
)
def _layer_kernel(t_hbm, src_hbm, dst_hbm, ew_hbm, ns_hbm, nd_hbm, out_hbm,
                  src_v, dst_v, c_v, ns_v, nd_v, grow_v, srow_v, agg_sh,
                  gs0, gs1, ss0, ss1):
    c = lax.axis_index("c")
    s = lax.axis_index("s")
    w = c * NS + s

    pltpu.sync_copy(src_hbm.at[w], src_v)
    pltpu.sync_copy(dst_hbm.at[w], dst_v)
    pltpu.sync_copy(ew_hbm.at[w], c_v)
    pltpu.sync_copy(ns_hbm, ns_v)
    pltpu.sync_copy(nd_hbm, nd_v)

    @pl.loop(0, CHUNK)
    def _(r):
        for q in range(HID // 16):
            srow_v[0, r, pl.ds(q * 16, 16)] = jnp.zeros((16,), jnp.float32)

    for r in range(RPS // CHUNK):
        pltpu.sync_copy(srow_v.at[0],
                        agg_sh.at[pl.ds(s * RPS + r * CHUNK, CHUNK)])

    @pl.loop(0, CPW)
    def _(j):
        @pl.loop(0, CHUNK, step=16)
        def _(e):
            sl = pl.ds(e, 16)
            si = src_v[j, sl]
            di = dst_v[j, sl]
            c_v[j, sl] = (c_v[j, sl]
                          * plsc.load_gather(ns_v, [si])
                          * plsc.load_gather(nd_v, [di]))

    plsc.subcore_barrier()

    gbufs = (grow_v.at[0], grow_v.at[1])
    sbufs = (srow_v.at[0], srow_v.at[1])
    gsems = (gs0, gs1)
    ssems = (ss0, ss1)

    iota16 = lax.iota(jnp.int32, 16)
    idx_even = iota16 * 2
    idx_odd = iota16 * 2 + 1

    pltpu.async_copy(t_hbm.at[src_v.at[0]], gbufs[0], gsems[0])
    pltpu.async_copy(t_hbm.at[src_v.at[1]], gbufs[1], gsems[1])

    @pl.loop(0, CPW, step=2)
    def _(j):
        for b in range(2):
            jj = j + b

            @pl.when(jj >= 2)
            def _():
                pltpu.make_async_copy(
                    sbufs[b], agg_sh.at[dst_v.at[jj - 2]], ssems[b]).wait()

            pltpu.make_async_copy(
                t_hbm.at[src_v.at[jj]], gbufs[b], gsems[b]).wait()

            @pl.loop(0, CHUNK, unroll=8)
            def _(e):
                w16 = plsc.load_gather(c_v, [_splat(jj), _splat(e)])
                for h in range(2):
                    v32 = gbufs[b][e, pl.ds(h * 32, 32)]
                    wi = plsc.bitcast(v32, jnp.int32)
                    lo = plsc.bitcast(wi << 16, jnp.float32) * w16
                    hi = plsc.bitcast(wi & jnp.int32(-65536), jnp.float32) * w16
                    plsc.store_scatter(srow_v,
                                       [_splat(b), _splat(e), idx_even + h * 32],
                                       lo)
                    plsc.store_scatter(srow_v,
                                       [_splat(b), _splat(e), idx_odd + h * 32],
                                       hi)

            pltpu.async_copy(sbufs[b], agg_sh.at[dst_v.at[jj]], ssems[b],
                             add=True)

            @pl.when(jj + 2 < CPW)
            def _():
                pltpu.async_copy(t_hbm.at[src_v.at[jj + 2]], gbufs[b],
                                 gsems[b])

    for b in range(2):
        pltpu.make_async_copy(
            sbufs[b], agg_sh.at[dst_v.at[CPW - 2 + b]], ssems[b]).wait()

    plsc.subcore_barrier()
    pltpu.sync_copy(agg_sh.at[pl.ds(s * RPS, RPS)],
                    out_hbm.at[c].at[pl.ds(s * RPS, RPS)])


def _tc_mm1(xp, w1):
    def body(x_ref, w_ref, t1_ref):
        t1_ref[...] = jnp.dot(x_ref[...], w_ref[...],
                              preferred_element_type=jnp.float32,
                              precision=lax.Precision.HIGHEST
                              ).astype(jnp.bfloat16)

    return pl.pallas_call(
        body,
        out_shape=jax.ShapeDtypeStruct((NP, HID), jnp.bfloat16),
    )(xp, w1)


def _tc_relu_mm(aggp, b, w2):
    def body(aggp_ref, b_ref, w_ref, t2_ref):
        h = jnp.maximum(aggp_ref[0] + aggp_ref[1] + b_ref[...][None, :], 0.0)
        t2_ref[...] = jnp.dot(h, w_ref[...],
                              preferred_element_type=jnp.float32,
                              precision=lax.Precision.HIGHEST
                              ).astype(jnp.bfloat16)

    return pl.pallas_call(
        body,
        out_shape=jax.ShapeDtypeStruct((NP, HID), jnp.bfloat16),
    )(aggp, b, w2)


def _tc_relu_out(aggp, b):
    def body(aggp_ref, b_ref, o_ref):
        h = aggp_ref[0] + aggp_ref[1] + b_ref[...][None, :]
        o_ref[...] = jnp.maximum(h[:N], 0.0)

    return pl.pallas_call(
        body,
        out_shape=jax.ShapeDtypeStruct((N, HID), jnp.float32),
    )(aggp, b)


def _pad_idx(a):
    a = a.reshape(NW, EPW)
    a = jnp.pad(a, ((0, 0), (0, CPW * CHUNK - EPW)), constant_values=N)
    return a.reshape(NW, CPW, CHUNK)


def _pad_ew(a):
    a = a.reshape(NW, EPW)
    a = jnp.pad(a, ((0, 0), (0, CPW * CHUNK - EPW)))
    return a.reshape(NW, CPW, CHUNK)


def kernel(x, edge_index0, edge_weight0, edge_index1, edge_weight1,
           W1, b1, W2, b2):
    src0 = _pad_idx(edge_index0[0])
    dst0 = _pad_idx(edge_index0[1])
    src1 = _pad_idx(edge_index1[0])
    dst1 = _pad_idx(edge_index1[1])
    ew0 = _pad_ew(edge_weight0)
    ew1 = _pad_ew(edge_weight1)
    xp = jnp.pad(x, ((0, NP - N), (0, 0)))

    idxs = jnp.stack([src0, dst0, src1, dst1])
    norms = _deg_kernel(idxs)
    t1 = _tc_mm1(xp, W1)
    aggp0 = _layer_kernel(t1, src0, dst0, ew0, norms[0], norms[1])
    t2 = _tc_relu_mm(aggp0, b1, W2)
    aggp1 = _layer_kernel(t2, src1, dst1, ew1, norms[2], norms[3])
    return _tc_relu_out(aggp1, b2)
